# baseline (speedup 1.0000x reference)
import functools

import jax
import jax.numpy as jnp
from jax import lax
from jax.experimental import pallas as pl
from jax.experimental.pallas import tpu as pltpu
from jax.experimental.pallas import tpu_sc as plsc

HIDDEN = 128
ID_DIM = 64
LANG_OFF = HIDDEN - ID_DIM
CHUNK = 80
NBUF = 8
AHEAD = 6
FUSE_ROWS = 16384


def _fuse_body(lang_ref, idt_ref, out_ref):
    out_ref[:, :LANG_OFF] = lang_ref[:, :LANG_OFF]
    out_ref[:, LANG_OFF:] = lang_ref[:, LANG_OFF:] + idt_ref[:, :].T


def _fuse_tables(language_table, id_table):
    V = language_table.shape[0]
    grid = pl.cdiv(V, FUSE_ROWS)
    idt = jnp.transpose(id_table)
    return pl.pallas_call(
        _fuse_body,
        grid=(grid,),
        in_specs=[
            pl.BlockSpec((FUSE_ROWS, HIDDEN), lambda i: (i, 0)),
            pl.BlockSpec((ID_DIM, FUSE_ROWS), lambda i: (0, i)),
        ],
        out_specs=pl.BlockSpec((FUSE_ROWS, HIDDEN), lambda i: (i, 0)),
        out_shape=jax.ShapeDtypeStruct((V, HIDDEN), jnp.float32),
    )(language_table, idt)


def kernel(item_ids, language_table, id_table):
    B, H = item_ids.shape
    N = B * H
    ids_flat = jnp.transpose(item_ids).reshape(N)
    fused = _fuse_tables(language_table, id_table)

    info = plsc.get_sparse_core_info()
    NC, NS = info.num_cores, info.num_subcores
    NW = NC * NS
    per_w = N // NW
    n_chunks = per_w // CHUNK
    n_groups = n_chunks // NBUF

    @functools.partial(
        pl.kernel,
        mesh=plsc.VectorSubcoreMesh(core_axis_name="c", subcore_axis_name="s"),
        out_type=jax.ShapeDtypeStruct((N, HIDDEN), jnp.float32),
        scratch_types=(
            [pltpu.VMEM((per_w,), jnp.int32)]
            + [pltpu.VMEM((CHUNK, HIDDEN), jnp.float32) for _ in range(NBUF)]
            + [pltpu.SemaphoreType.DMA for _ in range(2 * NBUF)]
        ),
        compiler_params=pltpu.CompilerParams(use_tc_tiling_on_sc=True),
    )
    def run(ids_hbm, tab_hbm, out_hbm, idx_all, *rest):
        bufs = rest[0:NBUF]
        g_sem = rest[NBUF:2 * NBUF]
        w_sem = rest[2 * NBUF:3 * NBUF]

        wid = lax.axis_index("s") * NC + lax.axis_index("c")
        base = wid * per_w
        pltpu.sync_copy(ids_hbm.at[pl.ds(base, per_w)], idx_all)

        def idx_slice(k):
            return idx_all.at[pl.ds(k * CHUNK, CHUNK)]

        def fire_gather(k, b):
            pltpu.async_copy(tab_hbm.at[idx_slice(k)], bufs[b], g_sem[b])

        def wait_gather(b):
            pltpu.make_async_copy(tab_hbm.at[idx_slice(0)], bufs[b], g_sem[b]).wait()

        def fire_write(k, b):
            pltpu.async_copy(
                bufs[b], out_hbm.at[pl.ds(base + k * CHUNK, CHUNK)], w_sem[b]
            )

        def wait_write(k, b):
            pltpu.make_async_copy(
                bufs[b], out_hbm.at[pl.ds(base + k * CHUNK, CHUNK)], w_sem[b]
            ).wait()

        for b in range(AHEAD):
            fire_gather(b, b)

        for b in range(NBUF):
            wait_gather(b)
            fire_write(b, b)
            kn = b + AHEAD
            if kn < NBUF:
                fire_gather(kn, kn % NBUF)
            else:
                bn = kn % NBUF
                wait_write(kn - NBUF, bn)
                fire_gather(kn, bn)

        def group_body(g, carry):
            for b in range(NBUF):
                k = g * NBUF + b
                wait_gather(b)
                fire_write(k, b)
                kn = k + AHEAD
                bn = (b + AHEAD) % NBUF

                @pl.when(kn < n_chunks)
                def _():
                    wait_write(kn - NBUF, bn)
                    fire_gather(kn, bn)

            return carry

        lax.fori_loop(1, n_groups, group_body, 0)

        for b in range(NBUF):
            wait_write(n_chunks - NBUF + b, (n_chunks - NBUF + b) % NBUF)

    out = run(ids_flat, fused)
    return jnp.transpose(out.reshape(H, B, HIDDEN), (1, 0, 2))


# speedup vs baseline: 1.0293x; 1.0293x over previous
import functools

import jax
import jax.numpy as jnp
from jax import lax
from jax.experimental import pallas as pl
from jax.experimental.pallas import tpu as pltpu
from jax.experimental.pallas import tpu_sc as plsc

HIDDEN = 128
ID_DIM = 64
LANG_OFF = HIDDEN - ID_DIM
NBUF = 5
AHEAD = 4
FUSE_ROWS = 16384


def _fuse_body(lang_ref, idt_ref, out_ref):
    out_ref[:, :LANG_OFF] = lang_ref[:, :LANG_OFF]
    out_ref[:, LANG_OFF:] = lang_ref[:, LANG_OFF:] + idt_ref[:, :].T


def _fuse_tables(language_table, id_table):
    V = language_table.shape[0]
    grid = pl.cdiv(V, FUSE_ROWS)
    idt = jnp.transpose(id_table)
    return pl.pallas_call(
        _fuse_body,
        grid=(grid,),
        in_specs=[
            pl.BlockSpec((FUSE_ROWS, HIDDEN), lambda i: (i, 0)),
            pl.BlockSpec((ID_DIM, FUSE_ROWS), lambda i: (0, i)),
        ],
        out_specs=pl.BlockSpec((FUSE_ROWS, HIDDEN), lambda i: (i, 0)),
        out_shape=jax.ShapeDtypeStruct((V, HIDDEN), jnp.float32),
    )(language_table, idt)


def kernel(item_ids, language_table, id_table):
    B, H = item_ids.shape
    N = B * H
    idst = jnp.transpose(item_ids)
    fused = _fuse_tables(language_table, id_table)

    info = plsc.get_sparse_core_info()
    NC, NS = info.num_cores, info.num_subcores
    NW = NC * NS
    cols = B // NW
    n_chunks = H
    n_groups = n_chunks // NBUF

    @functools.partial(
        pl.kernel,
        mesh=plsc.VectorSubcoreMesh(core_axis_name="c", subcore_axis_name="s"),
        out_type=jax.ShapeDtypeStruct((N, HIDDEN), jnp.float32),
        scratch_types=(
            [pltpu.VMEM((H, cols), jnp.int32)]
            + [pltpu.VMEM((cols, HIDDEN), jnp.float32) for _ in range(NBUF)]
            + [pltpu.SemaphoreType.DMA for _ in range(2 * NBUF)]
        ),
        compiler_params=pltpu.CompilerParams(use_tc_tiling_on_sc=True),
    )
    def run(ids_hbm, tab_hbm, out_hbm, idx_all, *rest):
        bufs = rest[0:NBUF]
        g_sem = rest[NBUF:2 * NBUF]
        w_sem = rest[2 * NBUF:3 * NBUF]

        wid = lax.axis_index("s") * NC + lax.axis_index("c")
        c0 = wid * cols
        pltpu.sync_copy(ids_hbm.at[:, pl.ds(c0, cols)], idx_all)

        def fire_gather(k, b):
            pltpu.async_copy(tab_hbm.at[idx_all.at[k]], bufs[b], g_sem[b])

        def wait_gather(b):
            pltpu.make_async_copy(tab_hbm.at[idx_all.at[0]], bufs[b], g_sem[b]).wait()

        def fire_write(k, b):
            pltpu.async_copy(
                bufs[b], out_hbm.at[pl.ds(k * B + c0, cols)], w_sem[b]
            )

        def wait_write(k, b):
            pltpu.make_async_copy(
                bufs[b], out_hbm.at[pl.ds(k * B + c0, cols)], w_sem[b]
            ).wait()

        for b in range(AHEAD):
            fire_gather(b, b)

        for b in range(NBUF):
            wait_gather(b)
            fire_write(b, b)
            kn = b + AHEAD
            if kn < NBUF:
                fire_gather(kn, kn % NBUF)
            else:
                bn = kn % NBUF
                wait_write(kn - NBUF, bn)
                fire_gather(kn, bn)

        def group_body(g, carry):
            for b in range(NBUF):
                k = g * NBUF + b
                wait_gather(b)
                fire_write(k, b)
                kn = k + AHEAD
                bn = (b + AHEAD) % NBUF

                @pl.when(kn < n_chunks)
                def _():
                    wait_write(kn - NBUF, bn)
                    fire_gather(kn, bn)

            return carry

        lax.fori_loop(1, n_groups, group_body, 0)

        for b in range(NBUF):
            wait_write(n_chunks - NBUF + b, (n_chunks - NBUF + b) % NBUF)

    out = run(idst, fused)
    return jnp.transpose(out.reshape(H, B, HIDDEN), (1, 0, 2))
